# Initial kernel scaffold; baseline (speedup 1.0000x reference)
import jax
import jax.numpy as jnp
from jax.experimental import pallas as pl


def kernel(x, edge_index, positive, W1, b1, W2, b2, W3, b3):
    raise NotImplementedError("write your pallas kernel here")


# speedup vs baseline: 47.4590x; 47.4590x over previous
import functools

import jax
import jax.numpy as jnp
from jax import lax
from jax.experimental import pallas as pl
from jax.experimental.pallas import tpu as pltpu
from jax.experimental.pallas import tpu_sc as plsc

F32 = jnp.float32
NC = 2
NS = 16
NW = NC * NS
CHUNK = 128
THRESHOLD = 0.0


def _round_up(v, m):
    return (v + m - 1) // m * m


@functools.lru_cache(maxsize=None)
def _sc_kernels(n_sent, nch, d):
    mesh = plsc.VectorSubcoreMesh(
        core_axis_name="c", subcore_axis_name="s",
        num_cores=NC, num_subcores=NS)
    zb = n_sent // NS
    half = nch // 2

    @functools.partial(
        pl.kernel,
        out_type=jax.ShapeDtypeStruct((NC * n_sent, d), F32),
        mesh=mesh,
        compiler_params=pltpu.CompilerParams(use_tc_tiling_on_sc=False),
        scratch_types=[
            pltpu.VMEM((nch, CHUNK), jnp.int32),
            pltpu.VMEM((nch, CHUNK), jnp.int32),
            pltpu.VMEM((2, CHUNK, d), F32),
            pltpu.VMEM((CHUNK, d), F32),
            pltpu.VMEM_SHARED((n_sent, d), F32),
            pltpu.SemaphoreType.DMA,
            pltpu.SemaphoreType.DMA,
        ],
    )
    def scatter_k(zs_hbm, srcw_hbm, dstw_hbm, out_hbm,
                  src_v, dst_v, rows_v, zrow_v, acc_sh, sem0, sem1):
        cid = lax.axis_index("c")
        sid = lax.axis_index("s")
        wid = cid * NS + sid
        zeros16 = jnp.zeros((16,), F32)
        pltpu.sync_copy(srcw_hbm.at[wid], src_v)
        pltpu.sync_copy(dstw_hbm.at[wid], dst_v)

        def zrow_body(i, carry):
            zrow_v[i, :] = zeros16
            return carry

        lax.fori_loop(0, CHUNK, zrow_body, 0)

        def zacc_body(q, carry):
            pltpu.sync_copy(
                zrow_v, acc_sh.at[pl.ds(sid * zb + q * CHUNK, CHUNK)])
            return carry

        lax.fori_loop(0, zb // CHUNK, zacc_body, 0)
        plsc.subcore_barrier()
        sems = (sem0, sem1)
        for b in range(2):
            pltpu.async_copy(zs_hbm.at[src_v.at[b]], rows_v.at[b], sems[b])

        def body(i, carry):
            for b in range(2):
                jj = 2 * i + b
                pltpu.make_async_copy(
                    zs_hbm.at[src_v.at[jj]], rows_v.at[b], sems[b]).wait()
                pltpu.sync_copy(rows_v.at[b], acc_sh.at[dst_v.at[jj]],
                                add=True)

                @pl.when(i + 1 < half)
                def _issue():
                    pltpu.async_copy(zs_hbm.at[src_v.at[jj + 2]],
                                     rows_v.at[b], sems[b])
            return carry

        lax.fori_loop(0, half, body, 0)
        plsc.subcore_barrier()
        base = cid * n_sent + sid * zb

        def wout_body(q, carry):
            pltpu.sync_copy(
                acc_sh.at[pl.ds(sid * zb + q * CHUNK, CHUNK)], zrow_v)
            pltpu.sync_copy(zrow_v, out_hbm.at[pl.ds(base + q * CHUNK,
                                                     CHUNK)])
            return carry

        lax.fori_loop(0, zb // CHUNK, wout_body, 0)

    @functools.partial(
        pl.kernel,
        out_type=jax.ShapeDtypeStruct((NC * n_sent,), F32),
        mesh=mesh,
        scratch_types=[
            pltpu.VMEM((nch, CHUNK), jnp.int32),
            pltpu.VMEM((CHUNK,), F32),
            pltpu.VMEM((zb,), F32),
            pltpu.VMEM_SHARED((n_sent,), F32),
        ],
    )
    def deg_k(dstw_hbm, out_hbm, dst_v, ones_v, zv, deg_sh):
        cid = lax.axis_index("c")
        sid = lax.axis_index("s")
        wid = cid * NS + sid
        pltpu.sync_copy(dstw_hbm.at[wid], dst_v)
        zeros16 = jnp.zeros((16,), F32)
        ones16 = jnp.ones((16,), F32)
        for kk in range(CHUNK // 16):
            ones_v[pl.ds(kk * 16, 16)] = ones16

        def zv_body(i, carry):
            zv[pl.ds(i * 16, 16)] = zeros16
            return carry

        lax.fori_loop(0, zb // 16, zv_body, 0)
        pltpu.sync_copy(zv, deg_sh.at[pl.ds(sid * zb, zb)])
        plsc.subcore_barrier()

        def body(j, carry):
            pltpu.sync_copy(ones_v, deg_sh.at[dst_v.at[j]], add=True)
            return carry

        lax.fori_loop(0, nch, body, 0)
        plsc.subcore_barrier()
        pltpu.sync_copy(deg_sh.at[pl.ds(sid * zb, zb)], zv)
        pltpu.sync_copy(zv, out_hbm.at[pl.ds(cid * n_sent + sid * zb, zb)])

    return scatter_k, deg_k


@functools.lru_cache(maxsize=None)
def _tc_kernels(n, n_sent, d_in, d, n_cls):
    def a_body(deg2_ref, x_ref, w1t_ref, dinv_ref, zs1_ref):
        deg = deg2_ref[0] + deg2_ref[1] + 1.0
        dinv = lax.rsqrt(deg)
        z = jnp.dot(x_ref[...], w1t_ref[...], preferred_element_type=F32)
        zfull = jnp.concatenate(
            [z, jnp.zeros((n_sent - n, d), F32)], axis=0)
        dinv_ref[...] = dinv
        zs1_ref[...] = zfull * dinv[:, None]

    tc_a = pl.pallas_call(
        a_body,
        out_shape=(jax.ShapeDtypeStruct((n_sent,), F32),
                   jax.ShapeDtypeStruct((n_sent, d), F32)),
    )

    def b_body(acc_ref, zs_ref, dinv_ref, b_ref, wt_ref, g_ref, zsn_ref):
        dinv = dinv_ref[...]
        s = (acc_ref[0] + acc_ref[1] + zs_ref[...]) * dinv[:, None] \
            + b_ref[...][None, :]
        h = jnp.maximum(s, 0.0)
        rid = lax.broadcasted_iota(jnp.int32, (n_sent, d), 0)
        h = jnp.where(rid < n, h, 0.0)
        g_ref[...] = (jnp.sum(h * h) * (1.0 / d)).reshape(1, 1)
        zsn_ref[...] = jnp.dot(h, wt_ref[...],
                               preferred_element_type=F32) * dinv[:, None]

    tc_b = pl.pallas_call(
        b_body,
        out_shape=(jax.ShapeDtypeStruct((1, 1), F32),
                   jax.ShapeDtypeStruct((n_sent, d), F32)),
    )

    def c_body(acc_ref, zs_ref, dinv_ref, b_ref, g1_ref, g2_ref, pos_ref,
               out_ref):
        dinv = dinv_ref[...]
        s = (acc_ref[0] + acc_ref[1] + zs_ref[...]) * dinv[:, None] \
            + b_ref[...][None, :]
        h = jnp.maximum(s, 0.0)
        rid = lax.broadcasted_iota(jnp.int32, (n_sent, d), 0)
        h = jnp.where(rid < n, h, 0.0)
        g3 = (jnp.sum(h * h) * (1.0 / n_cls)).reshape(1, 1)
        pos = pos_ref[...]

        def loss(g):
            p = jnp.where(
                g > 10.0 + THRESHOLD, 0.0,
                jnp.where(g < THRESHOLD - 10.0, THRESHOLD - g,
                          jnp.log(1.0 + jnp.exp(-g + THRESHOLD))))
            ng = jnp.where(
                g > 10.0 + THRESHOLD, THRESHOLD + g,
                jnp.where(g < THRESHOLD - 10.0, 0.0,
                          jnp.log(1.0 + jnp.exp(g + THRESHOLD))))
            return jnp.where(pos != 0, p, ng)

        out_ref[...] = loss(g1_ref[...]) + loss(g2_ref[...]) + loss(g3)

    tc_c = pl.pallas_call(
        c_body,
        out_shape=jax.ShapeDtypeStruct((1, 1), F32),
    )

    return tc_a, tc_b, tc_c


def kernel(x, edge_index, positive, W1, b1, W2, b2, W3, b3):
    n, d_in = x.shape
    e = edge_index.shape[1]
    d = W1.shape[0]
    n_cls = W3.shape[0]
    n_sent = _round_up(n + 16, NS * CHUNK)
    epw = _round_up(e, NW * CHUNK) // NW
    nch = epw // CHUNK
    tot = NW * epw
    npad = tot - e

    pad_idx = n + (jnp.arange(npad, dtype=edge_index.dtype) % (n_sent - n))
    srcw = jnp.concatenate([edge_index[0], pad_idx]).reshape(NW, nch, CHUNK)
    dstw = jnp.concatenate([edge_index[1], pad_idx]).reshape(NW, nch, CHUNK)
    scatter_k, deg_k = _sc_kernels(n_sent, nch, d)
    tc_a, tc_b, tc_c = _tc_kernels(n, n_sent, d_in, d, n_cls)

    deg2 = deg_k(dstw).reshape(NC, n_sent)
    dinv, zs1 = tc_a(deg2, x, W1.T)

    acc1 = scatter_k(zs1, srcw, dstw).reshape(NC, n_sent, d)
    g1, zs2 = tc_b(acc1, zs1, dinv, b1, W2.T)

    acc2 = scatter_k(zs2, srcw, dstw).reshape(NC, n_sent, d)
    w3t = jnp.zeros((d, d), F32).at[:, :n_cls].set(W3.T)
    g2, zs3 = tc_b(acc2, zs2, dinv, b2, w3t)

    acc3 = scatter_k(zs3, srcw, dstw).reshape(NC, n_sent, d)
    b3p = jnp.zeros((d,), F32).at[:n_cls].set(b3)
    pos = jnp.asarray(positive, jnp.int32).reshape(1, 1)
    out = tc_c(acc3, zs3, dinv, b3p, g1, g2, pos)
    return out[0, 0]


# speedup vs baseline: 62.1098x; 1.3087x over previous
import functools

import jax
import jax.numpy as jnp
from jax import lax
from jax.experimental import pallas as pl
from jax.experimental.pallas import tpu as pltpu
from jax.experimental.pallas import tpu_sc as plsc

F32 = jnp.float32
NC = 2
NS = 16
NW = NC * NS
CHUNK = 128
NBUF = 8
THRESHOLD = 0.0


def _round_up(v, m):
    return (v + m - 1) // m * m


@functools.lru_cache(maxsize=None)
def _sc_kernels(n_sent, nch, d):
    mesh = plsc.VectorSubcoreMesh(
        core_axis_name="c", subcore_axis_name="s",
        num_cores=NC, num_subcores=NS)
    zb = n_sent // NS
    nbuf = NBUF
    pref = 4
    assert nch % nbuf == 0

    @functools.partial(
        pl.kernel,
        out_type=jax.ShapeDtypeStruct((NC * n_sent, d), F32),
        mesh=mesh,
        compiler_params=pltpu.CompilerParams(use_tc_tiling_on_sc=False),
        scratch_types=[
            pltpu.VMEM((nch, CHUNK), jnp.int32),
            pltpu.VMEM((nch, CHUNK), jnp.int32),
            pltpu.VMEM((nbuf, CHUNK, d), F32),
            pltpu.VMEM((CHUNK, d), F32),
            pltpu.VMEM_SHARED((n_sent, d), F32),
            pltpu.SemaphoreType.DMA((nbuf,)),
            pltpu.SemaphoreType.DMA((nbuf,)),
        ],
    )
    def scatter_k(zs_hbm, srcw_hbm, dstw_hbm, out_hbm,
                  src_v, dst_v, rows_v, zrow_v, acc_sh, gsem, ssem):
        cid = lax.axis_index("c")
        sid = lax.axis_index("s")
        wid = cid * NS + sid
        zeros16 = jnp.zeros((16,), F32)
        pltpu.sync_copy(srcw_hbm.at[wid], src_v)
        pltpu.sync_copy(dstw_hbm.at[wid], dst_v)

        def zrow_body(i, carry):
            zrow_v[i, :] = zeros16
            return carry

        lax.fori_loop(0, CHUNK, zrow_body, 0)

        def zacc_body(q, carry):
            pltpu.sync_copy(
                zrow_v, acc_sh.at[pl.ds(sid * zb + q * CHUNK, CHUNK)])
            return carry

        lax.fori_loop(0, zb // CHUNK, zacc_body, 0)
        plsc.subcore_barrier()

        def g_start(jj, b):
            pltpu.async_copy(zs_hbm.at[src_v.at[jj]], rows_v.at[b],
                             gsem.at[b])

        def g_wait(jj, b):
            pltpu.make_async_copy(zs_hbm.at[src_v.at[jj]], rows_v.at[b],
                                  gsem.at[b]).wait()

        def s_start(jj, b):
            pltpu.async_copy(rows_v.at[b], acc_sh.at[dst_v.at[jj]],
                             ssem.at[b], add=True)

        def s_wait(jj, b):
            pltpu.make_async_copy(rows_v.at[b], acc_sh.at[dst_v.at[jj]],
                                  ssem.at[b]).wait()

        for b in range(pref):
            g_start(b, b)

        def body(i, carry):
            for b in range(nbuf):
                jj = nbuf * i + b
                g_wait(jj, b)
                s_start(jj, b)
                k = jj + pref

                @pl.when(k < nch)
                def _issue():
                    kb = (b + pref) % nbuf

                    @pl.when(k >= nbuf)
                    def _reuse():
                        s_wait(k - nbuf, kb)

                    g_start(k, kb)
            return carry

        lax.fori_loop(0, nch // nbuf, body, 0)
        for b in range(nbuf):
            s_wait(nch - nbuf + b, b)
        plsc.subcore_barrier()
        base = cid * n_sent + sid * zb

        def wout_body(q, carry):
            pltpu.sync_copy(
                acc_sh.at[pl.ds(sid * zb + q * CHUNK, CHUNK)], zrow_v)
            pltpu.sync_copy(zrow_v, out_hbm.at[pl.ds(base + q * CHUNK,
                                                     CHUNK)])
            return carry

        lax.fori_loop(0, zb // CHUNK, wout_body, 0)

    @functools.partial(
        pl.kernel,
        out_type=jax.ShapeDtypeStruct((NC * n_sent,), F32),
        mesh=mesh,
        scratch_types=[
            pltpu.VMEM((nch, CHUNK), jnp.int32),
            pltpu.VMEM((CHUNK,), F32),
            pltpu.VMEM((zb,), F32),
            pltpu.VMEM_SHARED((n_sent,), F32),
            pltpu.SemaphoreType.DMA,
        ],
    )
    def deg_k(dstw_hbm, out_hbm, dst_v, ones_v, zv, deg_sh, dsem):
        cid = lax.axis_index("c")
        sid = lax.axis_index("s")
        wid = cid * NS + sid
        pltpu.sync_copy(dstw_hbm.at[wid], dst_v)
        zeros16 = jnp.zeros((16,), F32)
        ones16 = jnp.ones((16,), F32)
        for kk in range(CHUNK // 16):
            ones_v[pl.ds(kk * 16, 16)] = ones16

        def zv_body(i, carry):
            zv[pl.ds(i * 16, 16)] = zeros16
            return carry

        lax.fori_loop(0, zb // 16, zv_body, 0)
        pltpu.sync_copy(zv, deg_sh.at[pl.ds(sid * zb, zb)])
        plsc.subcore_barrier()

        def body(j, carry):
            pltpu.async_copy(ones_v, deg_sh.at[dst_v.at[j]], dsem,
                             add=True)
            return carry

        lax.fori_loop(0, nch, body, 0)

        def drain(j, carry):
            pltpu.make_async_copy(ones_v, deg_sh.at[dst_v.at[j]],
                                  dsem).wait()
            return carry

        lax.fori_loop(0, nch, drain, 0)
        plsc.subcore_barrier()
        pltpu.sync_copy(deg_sh.at[pl.ds(sid * zb, zb)], zv)
        pltpu.sync_copy(zv, out_hbm.at[pl.ds(cid * n_sent + sid * zb, zb)])

    return scatter_k, deg_k


@functools.lru_cache(maxsize=None)
def _tc_kernels(n, n_sent, d_in, d, n_cls):
    def a_body(deg2_ref, x_ref, w1t_ref, dinv_ref, zs1_ref):
        deg = deg2_ref[0] + deg2_ref[1] + 1.0
        dinv = lax.rsqrt(deg)
        z = jnp.dot(x_ref[...], w1t_ref[...], preferred_element_type=F32)
        zfull = jnp.concatenate(
            [z, jnp.zeros((n_sent - n, d), F32)], axis=0)
        dinv_ref[...] = dinv
        zs1_ref[...] = zfull * dinv[:, None]

    tc_a = pl.pallas_call(
        a_body,
        out_shape=(jax.ShapeDtypeStruct((n_sent,), F32),
                   jax.ShapeDtypeStruct((n_sent, d), F32)),
    )

    def b_body(acc_ref, zs_ref, dinv_ref, b_ref, wt_ref, g_ref, zsn_ref):
        dinv = dinv_ref[...]
        s = (acc_ref[0] + acc_ref[1] + zs_ref[...]) * dinv[:, None] \
            + b_ref[...][None, :]
        h = jnp.maximum(s, 0.0)
        rid = lax.broadcasted_iota(jnp.int32, (n_sent, d), 0)
        h = jnp.where(rid < n, h, 0.0)
        g_ref[...] = (jnp.sum(h * h) * (1.0 / d)).reshape(1, 1)
        zsn_ref[...] = jnp.dot(h, wt_ref[...],
                               preferred_element_type=F32) * dinv[:, None]

    tc_b = pl.pallas_call(
        b_body,
        out_shape=(jax.ShapeDtypeStruct((1, 1), F32),
                   jax.ShapeDtypeStruct((n_sent, d), F32)),
    )

    def c_body(acc_ref, zs_ref, dinv_ref, b_ref, g1_ref, g2_ref, pos_ref,
               out_ref):
        dinv = dinv_ref[...]
        s = (acc_ref[0] + acc_ref[1] + zs_ref[...]) * dinv[:, None] \
            + b_ref[...][None, :]
        h = jnp.maximum(s, 0.0)
        rid = lax.broadcasted_iota(jnp.int32, (n_sent, d), 0)
        h = jnp.where(rid < n, h, 0.0)
        g3 = (jnp.sum(h * h) * (1.0 / n_cls)).reshape(1, 1)
        pos = pos_ref[...]

        def loss(g):
            p = jnp.where(
                g > 10.0 + THRESHOLD, 0.0,
                jnp.where(g < THRESHOLD - 10.0, THRESHOLD - g,
                          jnp.log(1.0 + jnp.exp(-g + THRESHOLD))))
            ng = jnp.where(
                g > 10.0 + THRESHOLD, THRESHOLD + g,
                jnp.where(g < THRESHOLD - 10.0, 0.0,
                          jnp.log(1.0 + jnp.exp(g + THRESHOLD))))
            return jnp.where(pos != 0, p, ng)

        out_ref[...] = loss(g1_ref[...]) + loss(g2_ref[...]) + loss(g3)

    tc_c = pl.pallas_call(
        c_body,
        out_shape=jax.ShapeDtypeStruct((1, 1), F32),
    )

    return tc_a, tc_b, tc_c


def kernel(x, edge_index, positive, W1, b1, W2, b2, W3, b3):
    n, d_in = x.shape
    e = edge_index.shape[1]
    d = W1.shape[0]
    n_cls = W3.shape[0]
    n_sent = _round_up(n + 16, NS * CHUNK)
    epw = _round_up(_round_up(e, NW) // NW, NBUF * CHUNK)
    nch = epw // CHUNK
    tot = NW * epw
    npad = tot - e

    pad_idx = n + (jnp.arange(npad, dtype=edge_index.dtype) % (n_sent - n))
    srcw = jnp.concatenate([edge_index[0], pad_idx]).reshape(NW, nch, CHUNK)
    dstw = jnp.concatenate([edge_index[1], pad_idx]).reshape(NW, nch, CHUNK)
    scatter_k, deg_k = _sc_kernels(n_sent, nch, d)
    tc_a, tc_b, tc_c = _tc_kernels(n, n_sent, d_in, d, n_cls)

    deg2 = deg_k(dstw).reshape(NC, n_sent)
    dinv, zs1 = tc_a(deg2, x, W1.T)

    acc1 = scatter_k(zs1, srcw, dstw).reshape(NC, n_sent, d)
    g1, zs2 = tc_b(acc1, zs1, dinv, b1, W2.T)

    acc2 = scatter_k(zs2, srcw, dstw).reshape(NC, n_sent, d)
    w3t = jnp.zeros((d, d), F32).at[:, :n_cls].set(W3.T)
    g2, zs3 = tc_b(acc2, zs2, dinv, b2, w3t)

    acc3 = scatter_k(zs3, srcw, dstw).reshape(NC, n_sent, d)
    b3p = jnp.zeros((d,), F32).at[:n_cls].set(b3)
    pos = jnp.asarray(positive, jnp.int32).reshape(1, 1)
    out = tc_c(acc3, zs3, dinv, b3p, g1, g2, pos)
    return out[0, 0]


# speedup vs baseline: 85.9657x; 1.3841x over previous
import functools

import jax
import jax.numpy as jnp
from jax import lax
from jax.experimental import pallas as pl
from jax.experimental.pallas import tpu as pltpu
from jax.experimental.pallas import tpu_sc as plsc

F32 = jnp.float32
NC = 2
NS = 16
NW = NC * NS
CHUNK = 128
NBUF = 8
THRESHOLD = 0.0


def _round_up(v, m):
    return (v + m - 1) // m * m


@functools.lru_cache(maxsize=None)
def _sc_kernels(n_sent, nch, d):
    mesh = plsc.VectorSubcoreMesh(
        core_axis_name="c", subcore_axis_name="s",
        num_cores=NC, num_subcores=NS)
    zb = n_sent // NS
    nbuf = NBUF
    pref = 4
    assert nch % nbuf == 0

    @functools.partial(
        pl.kernel,
        out_type=jax.ShapeDtypeStruct((NC * n_sent, d), F32),
        mesh=mesh,
        compiler_params=pltpu.CompilerParams(use_tc_tiling_on_sc=False),
        scratch_types=[
            pltpu.VMEM((nch, CHUNK), jnp.int32),
            pltpu.VMEM((nch, CHUNK), jnp.int32),
            pltpu.VMEM((nbuf, CHUNK, d), F32),
            pltpu.VMEM((CHUNK, d), F32),
            pltpu.VMEM_SHARED((n_sent, d), F32),
            pltpu.SemaphoreType.DMA((nbuf,)),
            pltpu.SemaphoreType.DMA((nbuf,)),
        ],
    )
    def scatter_k(zs_hbm, srcw_hbm, dstw_hbm, out_hbm,
                  src_v, dst_v, rows_v, zrow_v, acc_sh, gsem, ssem):
        cid = lax.axis_index("c")
        sid = lax.axis_index("s")
        wid = cid * NS + sid
        zeros16 = jnp.zeros((16,), F32)
        pltpu.sync_copy(srcw_hbm.at[wid], src_v)
        pltpu.sync_copy(dstw_hbm.at[wid], dst_v)

        def zrow_body(i, carry):
            zrow_v[i, :] = zeros16
            return carry

        lax.fori_loop(0, CHUNK, zrow_body, 0)

        def zacc_body(q, carry):
            pltpu.sync_copy(
                zrow_v, acc_sh.at[pl.ds(sid * zb + q * CHUNK, CHUNK)])
            return carry

        lax.fori_loop(0, zb // CHUNK, zacc_body, 0)
        plsc.subcore_barrier()

        def g_start(jj, b):
            pltpu.async_copy(zs_hbm.at[src_v.at[jj]], rows_v.at[b],
                             gsem.at[b])

        def g_wait(jj, b):
            pltpu.make_async_copy(zs_hbm.at[src_v.at[jj]], rows_v.at[b],
                                  gsem.at[b]).wait()

        def s_start(jj, b):
            pltpu.async_copy(rows_v.at[b], acc_sh.at[dst_v.at[jj]],
                             ssem.at[b], add=True)

        def s_wait(jj, b):
            pltpu.make_async_copy(rows_v.at[b], acc_sh.at[dst_v.at[jj]],
                                  ssem.at[b]).wait()

        for b in range(pref):
            g_start(b, b)

        def body(i, carry):
            for b in range(nbuf):
                jj = nbuf * i + b
                g_wait(jj, b)
                s_start(jj, b)
                k = jj + pref

                @pl.when(k < nch)
                def _issue():
                    kb = (b + pref) % nbuf

                    @pl.when(k >= nbuf)
                    def _reuse():
                        s_wait(k - nbuf, kb)

                    g_start(k, kb)
            return carry

        lax.fori_loop(0, nch // nbuf, body, 0)
        for b in range(nbuf):
            s_wait(nch - nbuf + b, b)
        plsc.subcore_barrier()
        base = cid * n_sent + sid * zb

        def wout_body(q, carry):
            pltpu.sync_copy(
                acc_sh.at[pl.ds(sid * zb + q * CHUNK, CHUNK)], zrow_v)
            pltpu.sync_copy(zrow_v, out_hbm.at[pl.ds(base + q * CHUNK,
                                                     CHUNK)])
            return carry

        lax.fori_loop(0, zb // CHUNK, wout_body, 0)

    @functools.partial(
        pl.kernel,
        out_type=jax.ShapeDtypeStruct((NC * n_sent,), F32),
        mesh=mesh,
        scratch_types=[
            pltpu.VMEM((nch, CHUNK), jnp.int32),
            pltpu.VMEM((CHUNK,), F32),
            pltpu.VMEM((zb,), F32),
            pltpu.VMEM_SHARED((n_sent,), F32),
            pltpu.SemaphoreType.DMA,
        ],
    )
    def deg_k(dstw_hbm, out_hbm, dst_v, ones_v, zv, deg_sh, dsem):
        cid = lax.axis_index("c")
        sid = lax.axis_index("s")
        wid = cid * NS + sid
        pltpu.sync_copy(dstw_hbm.at[wid], dst_v)
        zeros16 = jnp.zeros((16,), F32)
        ones16 = jnp.ones((16,), F32)
        for kk in range(CHUNK // 16):
            ones_v[pl.ds(kk * 16, 16)] = ones16

        def zv_body(i, carry):
            zv[pl.ds(i * 16, 16)] = zeros16
            return carry

        lax.fori_loop(0, zb // 16, zv_body, 0)
        pltpu.sync_copy(zv, deg_sh.at[pl.ds(sid * zb, zb)])
        plsc.subcore_barrier()

        def body(j, carry):
            pltpu.async_copy(ones_v, deg_sh.at[dst_v.at[j]], dsem,
                             add=True)
            return carry

        lax.fori_loop(0, nch, body, 0)

        def drain(j, carry):
            pltpu.make_async_copy(ones_v, deg_sh.at[dst_v.at[j]],
                                  dsem).wait()
            return carry

        lax.fori_loop(0, nch, drain, 0)
        plsc.subcore_barrier()
        pltpu.sync_copy(deg_sh.at[pl.ds(sid * zb, zb)], zv)
        pltpu.sync_copy(zv, out_hbm.at[pl.ds(cid * n_sent + sid * zb, zb)])

    return scatter_k, deg_k


@functools.lru_cache(maxsize=None)
def _tc_kernels(n, n_sent, d, n_cls):
    npk = n_sent * d // 128
    nxk = n // 8
    ndk = n_sent // 128

    def a_body(deg_ref, x_ref, bd1_ref, dinv_ref, z1_ref):
        deg = deg_ref[:ndk] + deg_ref[ndk:] + 1.0
        dinv_ref[...] = lax.rsqrt(deg)
        z1 = jnp.dot(x_ref[...], bd1_ref[...], preferred_element_type=F32)
        z1_ref[...] = jnp.concatenate(
            [z1, jnp.zeros((npk - nxk, 128), F32)], axis=0)

    tc_a = pl.pallas_call(
        a_body,
        out_shape=(jax.ShapeDtypeStruct((ndk, 128), F32),
                   jax.ShapeDtypeStruct((npk, 128), F32)),
    )

    def _mask_rows(h):
        rid = lax.broadcasted_iota(jnp.int32, (npk, 128), 0)
        eid = lax.broadcasted_iota(jnp.int32, (npk, 128), 1)
        node = rid * (128 // d) + eid // d
        return jnp.where(node < n, h, 0.0)

    def b_body(acc_ref, zs_ref, dinv_ref, b_ref, bd_ref, g_ref, zsn_ref):
        accs = acc_ref[:npk] + acc_ref[npk:]
        s = (accs + zs_ref[...]) * dinv_ref[...] + b_ref[...][None, :]
        h = _mask_rows(jnp.maximum(s, 0.0))
        g_ref[...] = (jnp.sum(h * h) * (1.0 / d)).reshape(1, 1)
        zsn_ref[...] = jnp.dot(h, bd_ref[...],
                               preferred_element_type=F32) * dinv_ref[...]

    tc_b = pl.pallas_call(
        b_body,
        out_shape=(jax.ShapeDtypeStruct((1, 1), F32),
                   jax.ShapeDtypeStruct((npk, 128), F32)),
    )

    def c_body(acc_ref, zs_ref, dinv_ref, b_ref, g1_ref, g2_ref, pos_ref,
               out_ref):
        accs = acc_ref[:npk] + acc_ref[npk:]
        s = (accs + zs_ref[...]) * dinv_ref[...] + b_ref[...][None, :]
        h = _mask_rows(jnp.maximum(s, 0.0))
        g3 = (jnp.sum(h * h) * (1.0 / n_cls)).reshape(1, 1)
        pos = pos_ref[...]

        def loss(gg):
            p = jnp.where(
                gg > 10.0 + THRESHOLD, 0.0,
                jnp.where(gg < THRESHOLD - 10.0, THRESHOLD - gg,
                          jnp.log(1.0 + jnp.exp(-gg + THRESHOLD))))
            ng = jnp.where(
                gg > 10.0 + THRESHOLD, THRESHOLD + gg,
                jnp.where(gg < THRESHOLD - 10.0, 0.0,
                          jnp.log(1.0 + jnp.exp(gg + THRESHOLD))))
            return jnp.where(pos != 0, p, ng)

        out_ref[...] = loss(g1_ref[...]) + loss(g2_ref[...]) + loss(g3)

    tc_c = pl.pallas_call(
        c_body,
        out_shape=jax.ShapeDtypeStruct((1, 1), F32),
    )

    return tc_a, tc_b, tc_c


def kernel(x, edge_index, positive, W1, b1, W2, b2, W3, b3):
    n, d_in = x.shape
    e = edge_index.shape[1]
    d = W1.shape[0]
    n_cls = W3.shape[0]
    n_sent = _round_up(n + 16, NS * CHUNK)
    epw = _round_up(_round_up(e, NW) // NW, NBUF * CHUNK)
    nch = epw // CHUNK
    tot = NW * epw
    npad = tot - e
    rep = 128 // d

    pad_idx = n + (jnp.arange(npad, dtype=edge_index.dtype) % (n_sent - n))
    srcw = jnp.concatenate([edge_index[0], pad_idx]).reshape(NW, nch, CHUNK)
    dstw = jnp.concatenate([edge_index[1], pad_idx]).reshape(NW, nch, CHUNK)

    scatter_k, deg_k = _sc_kernels(n_sent, nch, d)
    tc_a, tc_b, tc_c = _tc_kernels(n, n_sent, d, n_cls)

    eye = jnp.eye(rep, dtype=F32)
    bd1 = jnp.kron(eye, W1.T)
    bd2 = jnp.kron(eye, W2.T)
    w3t = jnp.zeros((d, d), F32).at[:, :n_cls].set(W3.T)
    bd3 = jnp.kron(eye, w3t)
    b1p = jnp.tile(b1, rep)
    b2p = jnp.tile(b2, rep)
    b3p = jnp.tile(jnp.zeros((d,), F32).at[:n_cls].set(b3), rep)
    x_pk = x.reshape(n // 8, 8 * d_in)

    degf = deg_k(dstw)
    dinv80, z1_pk = tc_a(degf.reshape(NC * n_sent // 128, 128), x_pk, bd1)

    dinv_rep = jnp.repeat(dinv80.reshape(n_sent), d).reshape(-1, 128)
    zs1_pk = z1_pk * dinv_rep

    npk = n_sent * d // 128
    acc1 = scatter_k(zs1_pk.reshape(n_sent, d), srcw, dstw)
    g1, zs2_pk = tc_b(acc1.reshape(2 * npk, 128), zs1_pk, dinv_rep,
                      b1p, bd2)

    acc2 = scatter_k(zs2_pk.reshape(n_sent, d), srcw, dstw)
    g2, zs3_pk = tc_b(acc2.reshape(2 * npk, 128), zs2_pk, dinv_rep,
                      b2p, bd3)

    acc3 = scatter_k(zs3_pk.reshape(n_sent, d), srcw, dstw)
    pos = jnp.asarray(positive, jnp.int32).reshape(1, 1)
    out = tc_c(acc3.reshape(2 * npk, 128), zs3_pk, dinv_rep, b3p,
               g1, g2, pos)
    return out[0, 0]


# speedup vs baseline: 95.7436x; 1.1137x over previous
import functools

import jax
import jax.numpy as jnp
from jax import lax
from jax.experimental import pallas as pl
from jax.experimental.pallas import tpu as pltpu
from jax.experimental.pallas import tpu_sc as plsc

F32 = jnp.float32
NC = 2
NS = 16
NW = NC * NS
CHUNK = 128
NBUF = 8
THRESHOLD = 0.0


def _round_up(v, m):
    return (v + m - 1) // m * m


@functools.lru_cache(maxsize=None)
def _sc_kernels(n_sent, nch, d):
    mesh = plsc.VectorSubcoreMesh(
        core_axis_name="c", subcore_axis_name="s",
        num_cores=NC, num_subcores=NS)
    zb = n_sent // NS
    nbuf = NBUF
    pref = 6
    assert nch % nbuf == 0

    @functools.partial(
        pl.kernel,
        out_type=jax.ShapeDtypeStruct((NC * n_sent, d), F32),
        mesh=mesh,
        compiler_params=pltpu.CompilerParams(use_tc_tiling_on_sc=False),
        scratch_types=[
            pltpu.VMEM((nch, CHUNK), jnp.int32),
            pltpu.VMEM((nch, CHUNK), jnp.int32),
            pltpu.VMEM((nbuf, CHUNK, d), F32),
            pltpu.VMEM((CHUNK, d), F32),
            pltpu.VMEM_SHARED((n_sent, d), F32),
            pltpu.SemaphoreType.DMA((nbuf,)),
            pltpu.SemaphoreType.DMA((nbuf,)),
        ],
    )
    def scatter_k(zs_hbm, srcw_hbm, dstw_hbm, out_hbm,
                  src_v, dst_v, rows_v, zrow_v, acc_sh, gsem, ssem):
        cid = lax.axis_index("c")
        sid = lax.axis_index("s")
        wid = cid * NS + sid
        zeros16 = jnp.zeros((16,), F32)
        pltpu.sync_copy(srcw_hbm.at[wid], src_v)
        pltpu.sync_copy(dstw_hbm.at[wid], dst_v)

        def zrow_body(i, carry):
            zrow_v[i, :] = zeros16
            return carry

        lax.fori_loop(0, CHUNK, zrow_body, 0)

        def zacc_body(q, carry):
            pltpu.sync_copy(
                zrow_v, acc_sh.at[pl.ds(sid * zb + q * CHUNK, CHUNK)])
            return carry

        lax.fori_loop(0, zb // CHUNK, zacc_body, 0)
        plsc.subcore_barrier()

        def g_start(jj, b):
            pltpu.async_copy(zs_hbm.at[src_v.at[jj]], rows_v.at[b],
                             gsem.at[b])

        def g_wait(jj, b):
            pltpu.make_async_copy(zs_hbm.at[src_v.at[jj]], rows_v.at[b],
                                  gsem.at[b]).wait()

        def s_start(jj, b):
            pltpu.async_copy(rows_v.at[b], acc_sh.at[dst_v.at[jj]],
                             ssem.at[b], add=True)

        def s_wait(jj, b):
            pltpu.make_async_copy(rows_v.at[b], acc_sh.at[dst_v.at[jj]],
                                  ssem.at[b]).wait()

        for b in range(pref):
            g_start(b, b)

        def body(i, carry):
            for b in range(nbuf):
                jj = nbuf * i + b
                g_wait(jj, b)
                s_start(jj, b)
                k = jj + pref

                @pl.when(k < nch)
                def _issue():
                    kb = (b + pref) % nbuf

                    @pl.when(k >= nbuf)
                    def _reuse():
                        s_wait(k - nbuf, kb)

                    g_start(k, kb)
            return carry

        lax.fori_loop(0, nch // nbuf, body, 0)
        for b in range(nbuf):
            s_wait(nch - nbuf + b, b)
        plsc.subcore_barrier()
        base = cid * n_sent + sid * zb

        def wout_body(q, carry):
            pltpu.sync_copy(
                acc_sh.at[pl.ds(sid * zb + q * CHUNK, CHUNK)], zrow_v)
            pltpu.sync_copy(zrow_v, out_hbm.at[pl.ds(base + q * CHUNK,
                                                     CHUNK)])
            return carry

        lax.fori_loop(0, zb // CHUNK, wout_body, 0)

    @functools.partial(
        pl.kernel,
        out_type=jax.ShapeDtypeStruct((NC * n_sent,), F32),
        mesh=mesh,
        scratch_types=[
            pltpu.VMEM((nch, CHUNK), jnp.int32),
            pltpu.VMEM((CHUNK,), F32),
            pltpu.VMEM((zb,), F32),
            pltpu.VMEM_SHARED((n_sent,), F32),
            pltpu.SemaphoreType.DMA,
        ],
    )
    def deg_k(dstw_hbm, out_hbm, dst_v, ones_v, zv, deg_sh, dsem):
        cid = lax.axis_index("c")
        sid = lax.axis_index("s")
        wid = cid * NS + sid
        pltpu.sync_copy(dstw_hbm.at[wid], dst_v)
        zeros16 = jnp.zeros((16,), F32)
        ones16 = jnp.ones((16,), F32)
        for kk in range(CHUNK // 16):
            ones_v[pl.ds(kk * 16, 16)] = ones16

        def zv_body(i, carry):
            zv[pl.ds(i * 16, 16)] = zeros16
            return carry

        lax.fori_loop(0, zb // 16, zv_body, 0)
        pltpu.sync_copy(zv, deg_sh.at[pl.ds(sid * zb, zb)])
        plsc.subcore_barrier()

        def body(j, carry):
            pltpu.async_copy(ones_v, deg_sh.at[dst_v.at[j]], dsem,
                             add=True)
            return carry

        lax.fori_loop(0, nch, body, 0)

        def drain(j, carry):
            pltpu.make_async_copy(ones_v, deg_sh.at[dst_v.at[j]],
                                  dsem).wait()
            return carry

        lax.fori_loop(0, nch, drain, 0)
        plsc.subcore_barrier()
        pltpu.sync_copy(deg_sh.at[pl.ds(sid * zb, zb)], zv)
        pltpu.sync_copy(zv, out_hbm.at[pl.ds(cid * n_sent + sid * zb, zb)])

    return scatter_k, deg_k


@functools.lru_cache(maxsize=None)
def _tc_kernels(n, n_sent, d, n_cls):
    npk = n_sent * d // 128
    nxk = n // 8
    ndk = n_sent // 128

    def a_body(deg_ref, x_ref, bd1_ref, dinv_ref, z1_ref):
        deg = deg_ref[:ndk] + deg_ref[ndk:] + 1.0
        dinv_ref[...] = lax.rsqrt(deg)
        z1 = jnp.dot(x_ref[...], bd1_ref[...], preferred_element_type=F32)
        z1_ref[...] = jnp.concatenate(
            [z1, jnp.zeros((npk - nxk, 128), F32)], axis=0)

    tc_a = pl.pallas_call(
        a_body,
        out_shape=(jax.ShapeDtypeStruct((ndk, 128), F32),
                   jax.ShapeDtypeStruct((npk, 128), F32)),
    )

    def _mask_rows(h):
        rid = lax.broadcasted_iota(jnp.int32, (npk, 128), 0)
        eid = lax.broadcasted_iota(jnp.int32, (npk, 128), 1)
        node = rid * (128 // d) + eid // d
        return jnp.where(node < n, h, 0.0)

    def b_body(acc_ref, zs_ref, dinv_ref, b_ref, bd_ref, g_ref, zsn_ref):
        accs = acc_ref[:npk] + acc_ref[npk:]
        s = (accs + zs_ref[...]) * dinv_ref[...] + b_ref[...][None, :]
        h = _mask_rows(jnp.maximum(s, 0.0))
        g_ref[...] = (jnp.sum(h * h) * (1.0 / d)).reshape(1, 1)
        zsn_ref[...] = jnp.dot(h, bd_ref[...],
                               preferred_element_type=F32) * dinv_ref[...]

    tc_b = pl.pallas_call(
        b_body,
        out_shape=(jax.ShapeDtypeStruct((1, 1), F32),
                   jax.ShapeDtypeStruct((npk, 128), F32)),
    )

    def c_body(acc_ref, zs_ref, dinv_ref, b_ref, g1_ref, g2_ref, pos_ref,
               out_ref):
        accs = acc_ref[:npk] + acc_ref[npk:]
        s = (accs + zs_ref[...]) * dinv_ref[...] + b_ref[...][None, :]
        h = _mask_rows(jnp.maximum(s, 0.0))
        g3 = (jnp.sum(h * h) * (1.0 / n_cls)).reshape(1, 1)
        pos = pos_ref[...]

        def loss(gg):
            p = jnp.where(
                gg > 10.0 + THRESHOLD, 0.0,
                jnp.where(gg < THRESHOLD - 10.0, THRESHOLD - gg,
                          jnp.log(1.0 + jnp.exp(-gg + THRESHOLD))))
            ng = jnp.where(
                gg > 10.0 + THRESHOLD, THRESHOLD + gg,
                jnp.where(gg < THRESHOLD - 10.0, 0.0,
                          jnp.log(1.0 + jnp.exp(gg + THRESHOLD))))
            return jnp.where(pos != 0, p, ng)

        out_ref[...] = loss(g1_ref[...]) + loss(g2_ref[...]) + loss(g3)

    tc_c = pl.pallas_call(
        c_body,
        out_shape=jax.ShapeDtypeStruct((1, 1), F32),
    )

    return tc_a, tc_b, tc_c


def kernel(x, edge_index, positive, W1, b1, W2, b2, W3, b3):
    n, d_in = x.shape
    e = edge_index.shape[1]
    d = W1.shape[0]
    n_cls = W3.shape[0]
    n_sent = _round_up(n + 16, NS * CHUNK)
    epw = _round_up(_round_up(e, NW) // NW, NBUF * CHUNK)
    nch = epw // CHUNK
    tot = NW * epw
    npad = tot - e
    rep = 128 // d

    pad_idx = n + (jnp.arange(npad, dtype=edge_index.dtype) % (n_sent - n))
    srcw = jnp.concatenate([edge_index[0], pad_idx]).reshape(NW, nch, CHUNK)
    dstw = jnp.concatenate([edge_index[1], pad_idx]).reshape(NW, nch, CHUNK)

    scatter_k, deg_k = _sc_kernels(n_sent, nch, d)
    tc_a, tc_b, tc_c = _tc_kernels(n, n_sent, d, n_cls)

    eye = jnp.eye(rep, dtype=F32)
    bd1 = jnp.kron(eye, W1.T)
    bd2 = jnp.kron(eye, W2.T)
    w3t = jnp.zeros((d, d), F32).at[:, :n_cls].set(W3.T)
    bd3 = jnp.kron(eye, w3t)
    b1p = jnp.tile(b1, rep)
    b2p = jnp.tile(b2, rep)
    b3p = jnp.tile(jnp.zeros((d,), F32).at[:n_cls].set(b3), rep)
    x_pk = x.reshape(n // 8, 8 * d_in)

    degf = deg_k(dstw)
    dinv80, z1_pk = tc_a(degf.reshape(NC * n_sent // 128, 128), x_pk, bd1)

    dinv_rep = jnp.repeat(dinv80.reshape(n_sent), d).reshape(-1, 128)
    zs1_pk = z1_pk * dinv_rep

    npk = n_sent * d // 128
    acc1 = scatter_k(zs1_pk.reshape(n_sent, d), srcw, dstw)
    g1, zs2_pk = tc_b(acc1.reshape(2 * npk, 128), zs1_pk, dinv_rep,
                      b1p, bd2)

    acc2 = scatter_k(zs2_pk.reshape(n_sent, d), srcw, dstw)
    g2, zs3_pk = tc_b(acc2.reshape(2 * npk, 128), zs2_pk, dinv_rep,
                      b2p, bd3)

    acc3 = scatter_k(zs3_pk.reshape(n_sent, d), srcw, dstw)
    pos = jnp.asarray(positive, jnp.int32).reshape(1, 1)
    out = tc_c(acc3.reshape(2 * npk, 128), zs3_pk, dinv_rep, b3p,
               g1, g2, pos)
    return out[0, 0]


# speedup vs baseline: 96.5442x; 1.0084x over previous
import functools

import jax
import jax.numpy as jnp
from jax import lax
from jax.experimental import pallas as pl
from jax.experimental.pallas import tpu as pltpu
from jax.experimental.pallas import tpu_sc as plsc

F32 = jnp.float32
NC = 2
NS = 16
NW = NC * NS
CHUNK = 128
NBUF = 8
THRESHOLD = 0.0


def _round_up(v, m):
    return (v + m - 1) // m * m


@functools.lru_cache(maxsize=None)
def _sc_kernels(n_sent, nch, d):
    mesh = plsc.VectorSubcoreMesh(
        core_axis_name="c", subcore_axis_name="s",
        num_cores=NC, num_subcores=NS)
    zb = n_sent // NS
    nbuf = NBUF
    pref = 7
    assert nch % nbuf == 0

    @functools.partial(
        pl.kernel,
        out_type=jax.ShapeDtypeStruct((NC * n_sent, d), F32),
        mesh=mesh,
        compiler_params=pltpu.CompilerParams(use_tc_tiling_on_sc=False),
        scratch_types=[
            pltpu.VMEM((nch, CHUNK), jnp.int32),
            pltpu.VMEM((nch, CHUNK), jnp.int32),
            pltpu.VMEM((nbuf, CHUNK, d), F32),
            pltpu.VMEM((CHUNK, d), F32),
            pltpu.VMEM_SHARED((n_sent, d), F32),
            pltpu.SemaphoreType.DMA((nbuf,)),
            pltpu.SemaphoreType.DMA((nbuf,)),
        ],
    )
    def scatter_k(zs_hbm, srcw_hbm, dstw_hbm, out_hbm,
                  src_v, dst_v, rows_v, zrow_v, acc_sh, gsem, ssem):
        cid = lax.axis_index("c")
        sid = lax.axis_index("s")
        wid = cid * NS + sid
        zeros16 = jnp.zeros((16,), F32)
        pltpu.sync_copy(srcw_hbm.at[wid], src_v)
        pltpu.sync_copy(dstw_hbm.at[wid], dst_v)

        def zrow_body(i, carry):
            zrow_v[i, :] = zeros16
            return carry

        lax.fori_loop(0, CHUNK, zrow_body, 0)

        def zacc_body(q, carry):
            pltpu.sync_copy(
                zrow_v, acc_sh.at[pl.ds(sid * zb + q * CHUNK, CHUNK)])
            return carry

        lax.fori_loop(0, zb // CHUNK, zacc_body, 0)
        plsc.subcore_barrier()

        def g_start(jj, b):
            pltpu.async_copy(zs_hbm.at[src_v.at[jj]], rows_v.at[b],
                             gsem.at[b])

        def g_wait(jj, b):
            pltpu.make_async_copy(zs_hbm.at[src_v.at[jj]], rows_v.at[b],
                                  gsem.at[b]).wait()

        def s_start(jj, b):
            pltpu.async_copy(rows_v.at[b], acc_sh.at[dst_v.at[jj]],
                             ssem.at[b], add=True)

        def s_wait(jj, b):
            pltpu.make_async_copy(rows_v.at[b], acc_sh.at[dst_v.at[jj]],
                                  ssem.at[b]).wait()

        for b in range(pref):
            g_start(b, b)

        def body(i, carry):
            for b in range(nbuf):
                jj = nbuf * i + b
                g_wait(jj, b)
                s_start(jj, b)
                k = jj + pref

                @pl.when(k < nch)
                def _issue():
                    kb = (b + pref) % nbuf

                    @pl.when(k >= nbuf)
                    def _reuse():
                        s_wait(k - nbuf, kb)

                    g_start(k, kb)
            return carry

        lax.fori_loop(0, nch // nbuf, body, 0)
        for b in range(nbuf):
            s_wait(nch - nbuf + b, b)
        plsc.subcore_barrier()
        base = cid * n_sent + sid * zb

        def wout_body(q, carry):
            pltpu.sync_copy(
                acc_sh.at[pl.ds(sid * zb + q * CHUNK, CHUNK)], zrow_v)
            pltpu.sync_copy(zrow_v, out_hbm.at[pl.ds(base + q * CHUNK,
                                                     CHUNK)])
            return carry

        lax.fori_loop(0, zb // CHUNK, wout_body, 0)

    @functools.partial(
        pl.kernel,
        out_type=jax.ShapeDtypeStruct((NC * n_sent,), F32),
        mesh=mesh,
        scratch_types=[
            pltpu.VMEM((nch, CHUNK), jnp.int32),
            pltpu.VMEM((CHUNK,), F32),
            pltpu.VMEM((zb,), F32),
            pltpu.VMEM_SHARED((n_sent,), F32),
            pltpu.SemaphoreType.DMA,
        ],
    )
    def deg_k(dstw_hbm, out_hbm, dst_v, ones_v, zv, deg_sh, dsem):
        cid = lax.axis_index("c")
        sid = lax.axis_index("s")
        wid = cid * NS + sid
        pltpu.sync_copy(dstw_hbm.at[wid], dst_v)
        zeros16 = jnp.zeros((16,), F32)
        ones16 = jnp.ones((16,), F32)
        for kk in range(CHUNK // 16):
            ones_v[pl.ds(kk * 16, 16)] = ones16

        def zv_body(i, carry):
            zv[pl.ds(i * 16, 16)] = zeros16
            return carry

        lax.fori_loop(0, zb // 16, zv_body, 0)
        pltpu.sync_copy(zv, deg_sh.at[pl.ds(sid * zb, zb)])
        plsc.subcore_barrier()

        def body(j, carry):
            pltpu.async_copy(ones_v, deg_sh.at[dst_v.at[j]], dsem,
                             add=True)
            return carry

        lax.fori_loop(0, nch, body, 0)

        def drain(j, carry):
            pltpu.make_async_copy(ones_v, deg_sh.at[dst_v.at[j]],
                                  dsem).wait()
            return carry

        lax.fori_loop(0, nch, drain, 0)
        plsc.subcore_barrier()
        pltpu.sync_copy(deg_sh.at[pl.ds(sid * zb, zb)], zv)
        pltpu.sync_copy(zv, out_hbm.at[pl.ds(cid * n_sent + sid * zb, zb)])

    return scatter_k, deg_k


@functools.lru_cache(maxsize=None)
def _tc_kernels(n, n_sent, d, n_cls):
    npk = n_sent * d // 128
    nxk = n // 8
    ndk = n_sent // 128

    def a_body(deg_ref, x_ref, bd1_ref, dinv_ref, z1_ref):
        deg = deg_ref[:ndk] + deg_ref[ndk:] + 1.0
        dinv_ref[...] = lax.rsqrt(deg)
        z1 = jnp.dot(x_ref[...], bd1_ref[...], preferred_element_type=F32)
        z1_ref[...] = jnp.concatenate(
            [z1, jnp.zeros((npk - nxk, 128), F32)], axis=0)

    tc_a = pl.pallas_call(
        a_body,
        out_shape=(jax.ShapeDtypeStruct((ndk, 128), F32),
                   jax.ShapeDtypeStruct((npk, 128), F32)),
    )

    def _mask_rows(h):
        rid = lax.broadcasted_iota(jnp.int32, (npk, 128), 0)
        eid = lax.broadcasted_iota(jnp.int32, (npk, 128), 1)
        node = rid * (128 // d) + eid // d
        return jnp.where(node < n, h, 0.0)

    def b_body(acc_ref, zs_ref, dinv_ref, b_ref, bd_ref, g_ref, zsn_ref):
        accs = acc_ref[:npk] + acc_ref[npk:]
        s = (accs + zs_ref[...]) * dinv_ref[...] + b_ref[...][None, :]
        h = _mask_rows(jnp.maximum(s, 0.0))
        g_ref[...] = (jnp.sum(h * h) * (1.0 / d)).reshape(1, 1)
        zsn_ref[...] = jnp.dot(h, bd_ref[...],
                               preferred_element_type=F32) * dinv_ref[...]

    tc_b = pl.pallas_call(
        b_body,
        out_shape=(jax.ShapeDtypeStruct((1, 1), F32),
                   jax.ShapeDtypeStruct((npk, 128), F32)),
    )

    def c_body(acc_ref, zs_ref, dinv_ref, b_ref, g1_ref, g2_ref, pos_ref,
               out_ref):
        accs = acc_ref[:npk] + acc_ref[npk:]
        s = (accs + zs_ref[...]) * dinv_ref[...] + b_ref[...][None, :]
        h = _mask_rows(jnp.maximum(s, 0.0))
        g3 = (jnp.sum(h * h) * (1.0 / n_cls)).reshape(1, 1)
        pos = pos_ref[...]

        def loss(gg):
            p = jnp.where(
                gg > 10.0 + THRESHOLD, 0.0,
                jnp.where(gg < THRESHOLD - 10.0, THRESHOLD - gg,
                          jnp.log(1.0 + jnp.exp(-gg + THRESHOLD))))
            ng = jnp.where(
                gg > 10.0 + THRESHOLD, THRESHOLD + gg,
                jnp.where(gg < THRESHOLD - 10.0, 0.0,
                          jnp.log(1.0 + jnp.exp(gg + THRESHOLD))))
            return jnp.where(pos != 0, p, ng)

        out_ref[...] = loss(g1_ref[...]) + loss(g2_ref[...]) + loss(g3)

    tc_c = pl.pallas_call(
        c_body,
        out_shape=jax.ShapeDtypeStruct((1, 1), F32),
    )

    return tc_a, tc_b, tc_c


def kernel(x, edge_index, positive, W1, b1, W2, b2, W3, b3):
    n, d_in = x.shape
    e = edge_index.shape[1]
    d = W1.shape[0]
    n_cls = W3.shape[0]
    n_sent = _round_up(n + 16, NS * CHUNK)
    epw = _round_up(_round_up(e, NW) // NW, NBUF * CHUNK)
    nch = epw // CHUNK
    tot = NW * epw
    npad = tot - e
    rep = 128 // d

    pad_idx = n + (jnp.arange(npad, dtype=edge_index.dtype) % (n_sent - n))
    srcw = jnp.concatenate([edge_index[0], pad_idx]).reshape(NW, nch, CHUNK)
    dstw = jnp.concatenate([edge_index[1], pad_idx]).reshape(NW, nch, CHUNK)

    scatter_k, deg_k = _sc_kernels(n_sent, nch, d)
    tc_a, tc_b, tc_c = _tc_kernels(n, n_sent, d, n_cls)

    eye = jnp.eye(rep, dtype=F32)
    bd1 = jnp.kron(eye, W1.T)
    bd2 = jnp.kron(eye, W2.T)
    w3t = jnp.zeros((d, d), F32).at[:, :n_cls].set(W3.T)
    bd3 = jnp.kron(eye, w3t)
    b1p = jnp.tile(b1, rep)
    b2p = jnp.tile(b2, rep)
    b3p = jnp.tile(jnp.zeros((d,), F32).at[:n_cls].set(b3), rep)
    x_pk = x.reshape(n // 8, 8 * d_in)

    degf = deg_k(dstw)
    dinv80, z1_pk = tc_a(degf.reshape(NC * n_sent // 128, 128), x_pk, bd1)

    dinv_rep = jnp.repeat(dinv80.reshape(n_sent), d).reshape(-1, 128)
    zs1_pk = z1_pk * dinv_rep

    npk = n_sent * d // 128
    acc1 = scatter_k(zs1_pk.reshape(n_sent, d), srcw, dstw)
    g1, zs2_pk = tc_b(acc1.reshape(2 * npk, 128), zs1_pk, dinv_rep,
                      b1p, bd2)

    acc2 = scatter_k(zs2_pk.reshape(n_sent, d), srcw, dstw)
    g2, zs3_pk = tc_b(acc2.reshape(2 * npk, 128), zs2_pk, dinv_rep,
                      b2p, bd3)

    acc3 = scatter_k(zs3_pk.reshape(n_sent, d), srcw, dstw)
    pos = jnp.asarray(positive, jnp.int32).reshape(1, 1)
    out = tc_c(acc3.reshape(2 * npk, 128), zs3_pk, dinv_rep, b3p,
               g1, g2, pos)
    return out[0, 0]


# speedup vs baseline: 98.2628x; 1.0178x over previous
import functools

import jax
import jax.numpy as jnp
from jax import lax
from jax.experimental import pallas as pl
from jax.experimental.pallas import tpu as pltpu
from jax.experimental.pallas import tpu_sc as plsc

F32 = jnp.float32
NC = 2
NS = 16
NW = NC * NS
CHUNK = 128
NBUF = 8
THRESHOLD = 0.0


def _round_up(v, m):
    return (v + m - 1) // m * m


@functools.lru_cache(maxsize=None)
def _sc_kernels(n_sent, nch, d):
    mesh = plsc.VectorSubcoreMesh(
        core_axis_name="c", subcore_axis_name="s",
        num_cores=NC, num_subcores=NS)
    zb = n_sent // NS
    nbuf = NBUF
    pref = 7
    assert nch % nbuf == 0

    @functools.partial(
        pl.kernel,
        out_type=jax.ShapeDtypeStruct((NC * n_sent, d), F32),
        mesh=mesh,
        compiler_params=pltpu.CompilerParams(use_tc_tiling_on_sc=False),
        scratch_types=[
            pltpu.VMEM((nch, CHUNK), jnp.int32),
            pltpu.VMEM((nch, CHUNK), jnp.int32),
            pltpu.VMEM((nbuf, CHUNK, d), F32),
            pltpu.VMEM((CHUNK, d), F32),
            pltpu.VMEM_SHARED((n_sent, d), F32),
            pltpu.SemaphoreType.DMA((nbuf,)),
            pltpu.SemaphoreType.DMA((nbuf,)),
            pltpu.SemaphoreType.DMA,
        ],
    )
    def scatter_k(zs_hbm, srcw_hbm, dstw_hbm, out_hbm,
                  src_v, dst_v, rows_v, zrow_v, acc_sh, gsem, ssem, zsem):
        cid = lax.axis_index("c")
        sid = lax.axis_index("s")
        wid = cid * NS + sid
        zeros16 = jnp.zeros((16,), F32)
        pltpu.sync_copy(srcw_hbm.at[wid], src_v)
        pltpu.sync_copy(dstw_hbm.at[wid], dst_v)

        def zrow_body(i, carry):
            zrow_v[i, :] = zeros16
            return carry

        lax.fori_loop(0, CHUNK, zrow_body, 0)

        def zacc_body(q, carry):
            pltpu.async_copy(
                zrow_v, acc_sh.at[pl.ds(sid * zb + q * CHUNK, CHUNK)],
                zsem)
            return carry

        lax.fori_loop(0, zb // CHUNK, zacc_body, 0)

        def zacc_drain(q, carry):
            pltpu.make_async_copy(
                zrow_v, acc_sh.at[pl.ds(sid * zb + q * CHUNK, CHUNK)],
                zsem).wait()
            return carry

        lax.fori_loop(0, zb // CHUNK, zacc_drain, 0)
        plsc.subcore_barrier()

        def g_start(jj, b):
            pltpu.async_copy(zs_hbm.at[src_v.at[jj]], rows_v.at[b],
                             gsem.at[b])

        def g_wait(jj, b):
            pltpu.make_async_copy(zs_hbm.at[src_v.at[jj]], rows_v.at[b],
                                  gsem.at[b]).wait()

        def s_start(jj, b):
            pltpu.async_copy(rows_v.at[b], acc_sh.at[dst_v.at[jj]],
                             ssem.at[b], add=True)

        def s_wait(jj, b):
            pltpu.make_async_copy(rows_v.at[b], acc_sh.at[dst_v.at[jj]],
                                  ssem.at[b]).wait()

        for b in range(pref):
            g_start(b, b)

        def body(i, carry):
            for b in range(nbuf):
                jj = nbuf * i + b
                g_wait(jj, b)
                s_start(jj, b)
                k = jj + pref

                @pl.when(k < nch)
                def _issue():
                    kb = (b + pref) % nbuf

                    @pl.when(k >= nbuf)
                    def _reuse():
                        s_wait(k - nbuf, kb)

                    g_start(k, kb)
            return carry

        lax.fori_loop(0, nch // nbuf, body, 0)
        for b in range(nbuf):
            s_wait(nch - nbuf + b, b)
        plsc.subcore_barrier()
        base = cid * n_sent + sid * zb
        wbufs = (zrow_v, rows_v.at[0])
        nq = zb // CHUNK
        for q in range(nq):
            wb = wbufs[q % 2]
            wsem = gsem.at[q % 2]
            if q >= 2:
                pltpu.make_async_copy(
                    wbufs[q % 2],
                    out_hbm.at[pl.ds(base + (q - 2) * CHUNK, CHUNK)],
                    wsem).wait()
            pltpu.sync_copy(
                acc_sh.at[pl.ds(sid * zb + q * CHUNK, CHUNK)], wb)
            pltpu.async_copy(
                wb, out_hbm.at[pl.ds(base + q * CHUNK, CHUNK)], wsem)
        for q in range(max(nq - 2, 0), nq):
            pltpu.make_async_copy(
                wbufs[q % 2],
                out_hbm.at[pl.ds(base + q * CHUNK, CHUNK)],
                gsem.at[q % 2]).wait()

    @functools.partial(
        pl.kernel,
        out_type=jax.ShapeDtypeStruct((NC * n_sent,), F32),
        mesh=mesh,
        scratch_types=[
            pltpu.VMEM((nch, CHUNK), jnp.int32),
            pltpu.VMEM((CHUNK,), F32),
            pltpu.VMEM((zb,), F32),
            pltpu.VMEM_SHARED((n_sent,), F32),
            pltpu.SemaphoreType.DMA,
        ],
    )
    def deg_k(dstw_hbm, out_hbm, dst_v, ones_v, zv, deg_sh, dsem):
        cid = lax.axis_index("c")
        sid = lax.axis_index("s")
        wid = cid * NS + sid
        pltpu.sync_copy(dstw_hbm.at[wid], dst_v)
        zeros16 = jnp.zeros((16,), F32)
        ones16 = jnp.ones((16,), F32)
        for kk in range(CHUNK // 16):
            ones_v[pl.ds(kk * 16, 16)] = ones16

        def zv_body(i, carry):
            zv[pl.ds(i * 16, 16)] = zeros16
            return carry

        lax.fori_loop(0, zb // 16, zv_body, 0)
        pltpu.sync_copy(zv, deg_sh.at[pl.ds(sid * zb, zb)])
        plsc.subcore_barrier()

        def body(j, carry):
            pltpu.async_copy(ones_v, deg_sh.at[dst_v.at[j]], dsem,
                             add=True)
            return carry

        lax.fori_loop(0, nch, body, 0)

        def drain(j, carry):
            pltpu.make_async_copy(ones_v, deg_sh.at[dst_v.at[j]],
                                  dsem).wait()
            return carry

        lax.fori_loop(0, nch, drain, 0)
        plsc.subcore_barrier()
        pltpu.sync_copy(deg_sh.at[pl.ds(sid * zb, zb)], zv)
        pltpu.sync_copy(zv, out_hbm.at[pl.ds(cid * n_sent + sid * zb, zb)])

    return scatter_k, deg_k


@functools.lru_cache(maxsize=None)
def _tc_kernels(n, n_sent, d, n_cls):
    npk = n_sent * d // 128
    nxk = n // 8
    ndk = n_sent // 128

    def a_body(deg_ref, x_ref, bd1_ref, dinv_ref, z1_ref):
        deg = deg_ref[:ndk] + deg_ref[ndk:] + 1.0
        dinv_ref[...] = lax.rsqrt(deg)
        z1 = jnp.dot(x_ref[...], bd1_ref[...], preferred_element_type=F32)
        z1_ref[...] = jnp.concatenate(
            [z1, jnp.zeros((npk - nxk, 128), F32)], axis=0)

    tc_a = pl.pallas_call(
        a_body,
        out_shape=(jax.ShapeDtypeStruct((ndk, 128), F32),
                   jax.ShapeDtypeStruct((npk, 128), F32)),
    )

    def _mask_rows(h):
        rid = lax.broadcasted_iota(jnp.int32, (npk, 128), 0)
        eid = lax.broadcasted_iota(jnp.int32, (npk, 128), 1)
        node = rid * (128 // d) + eid // d
        return jnp.where(node < n, h, 0.0)

    def b_body(acc_ref, zs_ref, dinv_ref, b_ref, bd_ref, g_ref, zsn_ref):
        accs = acc_ref[:npk] + acc_ref[npk:]
        s = (accs + zs_ref[...]) * dinv_ref[...] + b_ref[...][None, :]
        h = _mask_rows(jnp.maximum(s, 0.0))
        g_ref[...] = (jnp.sum(h * h) * (1.0 / d)).reshape(1, 1)
        zsn_ref[...] = jnp.dot(h, bd_ref[...],
                               preferred_element_type=F32) * dinv_ref[...]

    tc_b = pl.pallas_call(
        b_body,
        out_shape=(jax.ShapeDtypeStruct((1, 1), F32),
                   jax.ShapeDtypeStruct((npk, 128), F32)),
    )

    def c_body(acc_ref, zs_ref, dinv_ref, b_ref, g1_ref, g2_ref, pos_ref,
               out_ref):
        accs = acc_ref[:npk] + acc_ref[npk:]
        s = (accs + zs_ref[...]) * dinv_ref[...] + b_ref[...][None, :]
        h = _mask_rows(jnp.maximum(s, 0.0))
        g3 = (jnp.sum(h * h) * (1.0 / n_cls)).reshape(1, 1)
        pos = pos_ref[...]

        def loss(gg):
            p = jnp.where(
                gg > 10.0 + THRESHOLD, 0.0,
                jnp.where(gg < THRESHOLD - 10.0, THRESHOLD - gg,
                          jnp.log(1.0 + jnp.exp(-gg + THRESHOLD))))
            ng = jnp.where(
                gg > 10.0 + THRESHOLD, THRESHOLD + gg,
                jnp.where(gg < THRESHOLD - 10.0, 0.0,
                          jnp.log(1.0 + jnp.exp(gg + THRESHOLD))))
            return jnp.where(pos != 0, p, ng)

        out_ref[...] = loss(g1_ref[...]) + loss(g2_ref[...]) + loss(g3)

    tc_c = pl.pallas_call(
        c_body,
        out_shape=jax.ShapeDtypeStruct((1, 1), F32),
    )

    return tc_a, tc_b, tc_c


def kernel(x, edge_index, positive, W1, b1, W2, b2, W3, b3):
    n, d_in = x.shape
    e = edge_index.shape[1]
    d = W1.shape[0]
    n_cls = W3.shape[0]
    n_sent = _round_up(n + 16, NS * CHUNK)
    epw = _round_up(_round_up(e, NW) // NW, NBUF * CHUNK)
    nch = epw // CHUNK
    tot = NW * epw
    npad = tot - e
    rep = 128 // d

    pad_idx = n + (jnp.arange(npad, dtype=edge_index.dtype) % (n_sent - n))
    srcw = jnp.concatenate([edge_index[0], pad_idx]).reshape(NW, nch, CHUNK)
    dstw = jnp.concatenate([edge_index[1], pad_idx]).reshape(NW, nch, CHUNK)

    scatter_k, deg_k = _sc_kernels(n_sent, nch, d)
    tc_a, tc_b, tc_c = _tc_kernels(n, n_sent, d, n_cls)

    eye = jnp.eye(rep, dtype=F32)
    bd1 = jnp.kron(eye, W1.T)
    bd2 = jnp.kron(eye, W2.T)
    w3t = jnp.zeros((d, d), F32).at[:, :n_cls].set(W3.T)
    bd3 = jnp.kron(eye, w3t)
    b1p = jnp.tile(b1, rep)
    b2p = jnp.tile(b2, rep)
    b3p = jnp.tile(jnp.zeros((d,), F32).at[:n_cls].set(b3), rep)
    x_pk = x.reshape(n // 8, 8 * d_in)

    degf = deg_k(dstw)
    dinv80, z1_pk = tc_a(degf.reshape(NC * n_sent // 128, 128), x_pk, bd1)

    dinv_rep = jnp.repeat(dinv80.reshape(n_sent), d).reshape(-1, 128)
    zs1_pk = z1_pk * dinv_rep

    npk = n_sent * d // 128
    acc1 = scatter_k(zs1_pk.reshape(n_sent, d), srcw, dstw)
    g1, zs2_pk = tc_b(acc1.reshape(2 * npk, 128), zs1_pk, dinv_rep,
                      b1p, bd2)

    acc2 = scatter_k(zs2_pk.reshape(n_sent, d), srcw, dstw)
    g2, zs3_pk = tc_b(acc2.reshape(2 * npk, 128), zs2_pk, dinv_rep,
                      b2p, bd3)

    acc3 = scatter_k(zs3_pk.reshape(n_sent, d), srcw, dstw)
    pos = jnp.asarray(positive, jnp.int32).reshape(1, 1)
    out = tc_c(acc3.reshape(2 * npk, 128), zs3_pk, dinv_rep, b3p,
               g1, g2, pos)
    return out[0, 0]
